# Initial kernel scaffold; baseline (speedup 1.0000x reference)
import jax
import jax.numpy as jnp
from jax.experimental import pallas as pl


def kernel(x, edge_index, batch, emb, conv_Wh, conv_bh, conv_Wo, conv_bo, hid_W, hid_b, out_W, out_b):
    raise NotImplementedError("write your pallas kernel here")


# speedup vs baseline: 3.8538x; 3.8538x over previous
import jax
import jax.numpy as jnp
from jax import lax
from jax.experimental import pallas as pl
from jax.experimental.pallas import tpu as pltpu
from jax.experimental.pallas import tpu_sc as plsc

N = 50000
E = 800000
C = 64
CQ = 16
NSUB = 16
RPT = 3128
NO = RPT * NSUB
PADI = NO
NPH = NO + 8
EPT = 50176
EP = EPT * NSUB
ROWS2D = EP // 128
KMAC = 8
NMAC = EPT // (128 * KMAC)
RB = 400
NBLK = N // RB
EPS = 1e-5
NGRAPH = 64
HID = 256
NEG = -jnp.inf


def _segsum_body(h0, h1, h2, h3, g0, g1, zeros,
                 o0, o1, o2, o3, b0, b1, b2, b3,
                 idx_g, idx_s, rows, acc, sem):
    cid = lax.axis_index("c")
    sid = lax.axis_index("s")
    hq = (h0, h1, h2, h3)
    oq = ((o0, o1, o2, o3), (b0, b1, b2, b3))

    def do_pass(h_ref, o_ref, g2d, s2d):
        pltpu.sync_copy(zeros, acc.at[pl.ds(sid * RPT, RPT)])
        plsc.subcore_barrier()
        base = sid * (EPT // 128)

        def macro(m, _):
            r0 = base + m * KMAC
            pltpu.sync_copy(g2d.at[pl.ds(r0, KMAC)], idx_g)
            pltpu.sync_copy(s2d.at[pl.ds(r0, KMAC)], idx_s)
            cps = [pltpu.async_copy(h_ref.at[idx_g.at[j]], rows.at[j], sem)
                   for j in range(KMAC)]
            for cp in cps:
                cp.wait()
            for j in range(KMAC):
                pltpu.sync_copy(rows.at[j], acc.at[idx_s.at[j]], add=True)
            return 0

        lax.fori_loop(0, NMAC, macro, 0)
        plsc.subcore_barrier()
        pltpu.sync_copy(acc.at[pl.ds(sid * RPT, RPT)],
                        o_ref.at[pl.ds(sid * RPT, RPT)])
        plsc.subcore_barrier()

    for p in range(2):
        g2d = g0 if p == 0 else g1
        s2d = g1 if p == 0 else g0
        for q in range(2):
            @pl.when(cid == 0)
            def _(p=p, q=q, g2d=g2d, s2d=s2d):
                do_pass(hq[q], oq[p][q], g2d, s2d)

            @pl.when(cid == 1)
            def _(p=p, q=q, g2d=g2d, s2d=s2d):
                do_pass(hq[2 + q], oq[p][2 + q], g2d, s2d)


_segsum = pl.kernel(
    _segsum_body,
    out_type=[jax.ShapeDtypeStruct((NO, CQ), jnp.float32)] * 8,
    mesh=plsc.VectorSubcoreMesh(core_axis_name="c", subcore_axis_name="s"),
    scratch_types=[
        pltpu.VMEM((KMAC, 128), jnp.int32),
        pltpu.VMEM((KMAC, 128), jnp.int32),
        pltpu.VMEM((KMAC, 128, CQ), jnp.float32),
        pltpu.VMEM_SHARED((NPH, CQ), jnp.float32),
        pltpu.SemaphoreType.DMA,
    ],
    compiler_params=pltpu.CompilerParams(use_tc_tiling_on_sc=False),
)


def _embed_body(xc_ref, emb_ref, *outs):
    xc = xc_ref[...]
    acc = jnp.zeros((RB, C), jnp.float32)
    for t in range(6):
        acc += jnp.where(xc == t, 1.0, 0.0) * emb_ref[t][None, :]
    for q in range(4):
        outs[q][...] = acc[:, q * CQ:(q + 1) * CQ]


_embed = pl.pallas_call(
    _embed_body,
    grid=(NBLK,),
    in_specs=[
        pl.BlockSpec((RB, 1), lambda i: (i, 0)),
        pl.BlockSpec((6, C), lambda i: (0, 0)),
    ],
    out_specs=[pl.BlockSpec((RB, CQ), lambda i: (i, 0))] * 4,
    out_shape=[jax.ShapeDtypeStruct((NPH, CQ), jnp.float32)] * 4,
)


def _mlp_body(*refs):
    outs = refs[:8]
    Wh, bh, Wo, bo = refs[8:12]
    c2_ref, s_ref = refs[12:14]
    i = pl.program_id(0)
    cc = jnp.concatenate([r[...] for r in outs], axis=1)
    hid = jnp.maximum(
        jnp.dot(cc, Wh[...], preferred_element_type=jnp.float32) + bh[...], 0.0)
    c2 = jnp.dot(hid, Wo[...], preferred_element_type=jnp.float32) + bo[...]
    c2_ref[...] = c2

    @pl.when(i == 0)
    def _():
        s_ref[...] = jnp.zeros_like(s_ref)

    s_ref[0:1, :] += jnp.sum(c2, axis=0, keepdims=True)
    s_ref[1:2, :] += jnp.sum(c2 * c2, axis=0, keepdims=True)


_mlp = pl.pallas_call(
    _mlp_body,
    grid=(NBLK,),
    in_specs=[pl.BlockSpec((RB, CQ), lambda i: (i, 0))] * 8 + [
        pl.BlockSpec((2 * C, HID), lambda i: (0, 0)),
        pl.BlockSpec((1, HID), lambda i: (0, 0)),
        pl.BlockSpec((HID, C), lambda i: (0, 0)),
        pl.BlockSpec((1, C), lambda i: (0, 0)),
    ],
    out_specs=[
        pl.BlockSpec((RB, C), lambda i: (i, 0)),
        pl.BlockSpec((8, C), lambda i: (0, 0)),
    ],
    out_shape=[
        jax.ShapeDtypeStruct((N, C), jnp.float32),
        jax.ShapeDtypeStruct((8, C), jnp.float32),
    ],
)


def _norm_body(*refs):
    hs = refs[:4]
    c2_ref, s_ref = refs[4:6]
    outs = refs[6:10]
    s = s_ref[...]
    mean = s[0:1, :] * (1.0 / N)
    var = s[1:2, :] * (1.0 / N) - mean * mean
    inv = lax.rsqrt(var + EPS)
    h = jnp.concatenate([r[...] for r in hs], axis=1)
    hn = jnp.maximum(h + (c2_ref[...] - mean) * inv, 0.0)
    for q in range(4):
        outs[q][...] = hn[:, q * CQ:(q + 1) * CQ]


_norm = pl.pallas_call(
    _norm_body,
    grid=(NBLK,),
    in_specs=[pl.BlockSpec((RB, CQ), lambda i: (i, 0))] * 4 + [
        pl.BlockSpec((RB, C), lambda i: (i, 0)),
        pl.BlockSpec((8, C), lambda i: (0, 0)),
    ],
    out_specs=[pl.BlockSpec((RB, CQ), lambda i: (i, 0))] * 4,
    out_shape=[jax.ShapeDtypeStruct((NPH, CQ), jnp.float32)] * 4,
)


def _pool_body(*refs):
    bcol_ref = refs[0]
    hs = refs[1:37]
    out = refs[37]
    i = pl.program_id(0)

    @pl.when(i == 0)
    def _():
        out[...] = jnp.full_like(out, NEG)

    xcat = jnp.concatenate([r[...] for r in hs], axis=1)
    b = bcol_ref[...]
    lo = jnp.min(b)
    hi = jnp.max(b)

    def body(g, _):
        @pl.when(jnp.logical_and(g >= lo, g <= hi))
        def _():
            bm = jnp.max(jnp.where(b == g, xcat, NEG), axis=0, keepdims=True)
            out[pl.ds(g, 1), :] = jnp.maximum(out[pl.ds(g, 1), :], bm)
        return 0

    lax.fori_loop(0, NGRAPH, body, 0)


_pool = pl.pallas_call(
    _pool_body,
    grid=(NBLK,),
    in_specs=[pl.BlockSpec((RB, 1), lambda i: (i, 0))]
    + [pl.BlockSpec((RB, CQ), lambda i: (i, 0))] * 36,
    out_specs=pl.BlockSpec((NGRAPH, 9 * C), lambda i: (0, 0)),
    out_shape=jax.ShapeDtypeStruct((NGRAPH, 9 * C), jnp.float32),
)


def _head_body(p_ref, hW_ref, hb_ref, oW_ref, ob_ref, out_ref):
    z = jnp.maximum(
        jnp.dot(p_ref[...], hW_ref[...], preferred_element_type=jnp.float32)
        + hb_ref[...], 0.0)
    out_ref[...] = (jnp.dot(z, oW_ref[...], preferred_element_type=jnp.float32)
                    + ob_ref[...])


_head = pl.pallas_call(
    _head_body,
    out_shape=jax.ShapeDtypeStruct((NGRAPH, 1), jnp.float32),
)


def kernel(x, edge_index, batch, emb, conv_Wh, conv_bh, conv_Wo, conv_bo,
           hid_W, hid_b, out_W, out_b):
    src = edge_index[0].astype(jnp.int32)
    dst = edge_index[1].astype(jnp.int32)
    pad = jnp.full((EP - E,), PADI, jnp.int32)
    src2d = jnp.concatenate([src, pad]).reshape(ROWS2D, 128)
    dst2d = jnp.concatenate([dst, pad]).reshape(ROWS2D, 128)
    zeros = jnp.zeros((RPT, CQ), jnp.float32)

    xcol = x.reshape(N, 1).astype(jnp.int32)
    hq = _embed(xcol, emb)
    parts = list(hq)
    for i in range(8):
        segs = _segsum(*hq, src2d, dst2d, zeros)
        c2, sums = _mlp(*segs,
                        conv_Wh[i], conv_bh[i].reshape(1, HID),
                        conv_Wo[i], conv_bo[i].reshape(1, C))
        hq = _norm(*hq, c2, sums)
        parts.extend(hq)

    bcol = batch.reshape(N, 1).astype(jnp.int32)
    pooled = _pool(bcol, *parts)
    r = _head(pooled, hid_W, hid_b.reshape(1, -1), out_W, out_b.reshape(1, 1))
    return r.reshape(-1)


# speedup vs baseline: 5.5006x; 1.4273x over previous
import jax
import jax.numpy as jnp
from jax import lax
from jax.experimental import pallas as pl
from jax.experimental.pallas import tpu as pltpu
from jax.experimental.pallas import tpu_sc as plsc

N = 50000
E = 800000
C = 64
CQ = 16
NSUB = 16
RPT = 3128
NO = RPT * NSUB
PADI = NO
NPH = NO + 8
EPT = 50176
EP = EPT * NSUB
ROWS2D = EP // 128
GIR = 7
GB = GIR * 128
MH = 28
NGRP = ROWS2D // GIR
RB = 400
NBLK = N // RB
EPS = 1e-5
NGRAPH = 64
HID = 256
NEG = -jnp.inf


def _segsum_body(h0, h1, h2, h3, g0, g1, zeros,
                 o0, o1, o2, o3, b0, b1, b2, b3,
                 idx_g, idx_s, rows, acc,
                 semg0, semg1, sems0, sems1, semi0, semi1):
    cid = lax.axis_index("c")
    sid = lax.axis_index("s")
    hq = (h0, h1, h2, h3)
    oq = ((o0, o1, o2, o3), (b0, b1, b2, b3))
    semg = (semg0, semg1)
    sems = (sems0, sems1)
    semi = (semi0, semi1)

    def do_pass(h_ref, o_ref, g3d, s3d):
        pltpu.sync_copy(zeros, acc.at[pl.ds(sid * RPT, RPT)])
        plsc.subcore_barrier()

        def load_idx(b, grp):
            pltpu.async_copy(g3d.at[grp], idx_g.at[b], semi[b])
            pltpu.async_copy(s3d.at[grp], idx_s.at[b], semi[b])

        def wait_idx(b):
            pltpu.make_async_copy(g3d.at[0], idx_g.at[b], semi[b]).wait()
            pltpu.make_async_copy(s3d.at[0], idx_s.at[b], semi[b]).wait()

        def fire_g(b):
            for j in range(GIR):
                pltpu.async_copy(h_ref.at[idx_g.at[b, j]],
                                 rows.at[b, pl.ds(j * 128, 128)], semg[b])

        def fire_s(b):
            for j in range(GIR):
                pltpu.async_copy(rows.at[b, pl.ds(j * 128, 128)],
                                 acc.at[idx_s.at[b, j]], sems[b], add=True)

        def drain(sem):
            pltpu.make_async_copy(zeros.at[pl.ds(0, GB)],
                                  rows.at[0], sem).wait()

        for half in range(2):
            grp0 = sid * (2 * MH) + half * MH
            load_idx(0, grp0)
            load_idx(1, grp0 + 1)
            wait_idx(0)
            fire_g(0)
            wait_idx(1)
            fire_g(1)

            def body(i, _):
                for b in range(2):
                    m = 2 * i + b
                    drain(semg[b])
                    fire_s(b)

                    @pl.when(m + 2 < MH)
                    def _(b=b, m=m):
                        drain(sems[b])
                        load_idx(b, grp0 + m + 2)
                        wait_idx(b)
                        fire_g(b)
                return 0

            lax.fori_loop(0, MH // 2, body, 0)
            drain(sems[0])
            drain(sems[1])

        plsc.subcore_barrier()
        pltpu.sync_copy(acc.at[pl.ds(sid * RPT, RPT)],
                        o_ref.at[pl.ds(sid * RPT, RPT)])
        plsc.subcore_barrier()

    for p in range(2):
        g2d = g0 if p == 0 else g1
        s2d = g1 if p == 0 else g0
        for q in range(2):
            @pl.when(cid == 0)
            def _(p=p, q=q, g2d=g2d, s2d=s2d):
                do_pass(hq[q], oq[p][q], g2d, s2d)

            @pl.when(cid == 1)
            def _(p=p, q=q, g2d=g2d, s2d=s2d):
                do_pass(hq[2 + q], oq[p][2 + q], g2d, s2d)


_segsum = pl.kernel(
    _segsum_body,
    out_type=[jax.ShapeDtypeStruct((NO, CQ), jnp.float32)] * 8,
    mesh=plsc.VectorSubcoreMesh(core_axis_name="c", subcore_axis_name="s"),
    scratch_types=[
        pltpu.VMEM((2, GIR, 128), jnp.int32),
        pltpu.VMEM((2, GIR, 128), jnp.int32),
        pltpu.VMEM((2, GB, CQ), jnp.float32),
        pltpu.VMEM_SHARED((NPH, CQ), jnp.float32),
        pltpu.SemaphoreType.DMA,
        pltpu.SemaphoreType.DMA,
        pltpu.SemaphoreType.DMA,
        pltpu.SemaphoreType.DMA,
        pltpu.SemaphoreType.DMA,
        pltpu.SemaphoreType.DMA,
    ],
    compiler_params=pltpu.CompilerParams(use_tc_tiling_on_sc=False),
)


def _embed_body(xc_ref, emb_ref, *outs):
    xc = xc_ref[...]
    acc = jnp.zeros((RB, C), jnp.float32)
    for t in range(6):
        acc += jnp.where(xc == t, 1.0, 0.0) * emb_ref[t][None, :]
    for q in range(4):
        outs[q][...] = acc[:, q * CQ:(q + 1) * CQ]


_embed = pl.pallas_call(
    _embed_body,
    grid=(NBLK,),
    in_specs=[
        pl.BlockSpec((RB, 1), lambda i: (i, 0)),
        pl.BlockSpec((6, C), lambda i: (0, 0)),
    ],
    out_specs=[pl.BlockSpec((RB, CQ), lambda i: (i, 0))] * 4,
    out_shape=[jax.ShapeDtypeStruct((NPH, CQ), jnp.float32)] * 4,
)


def _mlp_body(*refs):
    outs = refs[:8]
    Wh, bh, Wo, bo = refs[8:12]
    c2_ref, s_ref = refs[12:14]
    i = pl.program_id(0)
    cc = jnp.concatenate([r[...] for r in outs], axis=1)
    hid = jnp.maximum(
        jnp.dot(cc, Wh[...], preferred_element_type=jnp.float32) + bh[...], 0.0)
    c2 = jnp.dot(hid, Wo[...], preferred_element_type=jnp.float32) + bo[...]
    c2_ref[...] = c2

    @pl.when(i == 0)
    def _():
        s_ref[...] = jnp.zeros_like(s_ref)

    s_ref[0:1, :] += jnp.sum(c2, axis=0, keepdims=True)
    s_ref[1:2, :] += jnp.sum(c2 * c2, axis=0, keepdims=True)


_mlp = pl.pallas_call(
    _mlp_body,
    grid=(NBLK,),
    in_specs=[pl.BlockSpec((RB, CQ), lambda i: (i, 0))] * 8 + [
        pl.BlockSpec((2 * C, HID), lambda i: (0, 0)),
        pl.BlockSpec((1, HID), lambda i: (0, 0)),
        pl.BlockSpec((HID, C), lambda i: (0, 0)),
        pl.BlockSpec((1, C), lambda i: (0, 0)),
    ],
    out_specs=[
        pl.BlockSpec((RB, C), lambda i: (i, 0)),
        pl.BlockSpec((8, C), lambda i: (0, 0)),
    ],
    out_shape=[
        jax.ShapeDtypeStruct((N, C), jnp.float32),
        jax.ShapeDtypeStruct((8, C), jnp.float32),
    ],
)


def _norm_body(*refs):
    hs = refs[:4]
    c2_ref, s_ref = refs[4:6]
    outs = refs[6:10]
    s = s_ref[...]
    mean = s[0:1, :] * (1.0 / N)
    var = s[1:2, :] * (1.0 / N) - mean * mean
    inv = lax.rsqrt(var + EPS)
    h = jnp.concatenate([r[...] for r in hs], axis=1)
    hn = jnp.maximum(h + (c2_ref[...] - mean) * inv, 0.0)
    for q in range(4):
        outs[q][...] = hn[:, q * CQ:(q + 1) * CQ]


_norm = pl.pallas_call(
    _norm_body,
    grid=(NBLK,),
    in_specs=[pl.BlockSpec((RB, CQ), lambda i: (i, 0))] * 4 + [
        pl.BlockSpec((RB, C), lambda i: (i, 0)),
        pl.BlockSpec((8, C), lambda i: (0, 0)),
    ],
    out_specs=[pl.BlockSpec((RB, CQ), lambda i: (i, 0))] * 4,
    out_shape=[jax.ShapeDtypeStruct((NPH, CQ), jnp.float32)] * 4,
)


def _pool_body(*refs):
    bcol_ref = refs[0]
    hs = refs[1:37]
    out = refs[37]
    i = pl.program_id(0)

    @pl.when(i == 0)
    def _():
        out[...] = jnp.full_like(out, NEG)

    xcat = jnp.concatenate([r[...] for r in hs], axis=1)
    b = bcol_ref[...]
    lo = jnp.min(b)
    hi = jnp.max(b)

    def body(g, _):
        @pl.when(jnp.logical_and(g >= lo, g <= hi))
        def _():
            bm = jnp.max(jnp.where(b == g, xcat, NEG), axis=0, keepdims=True)
            out[pl.ds(g, 1), :] = jnp.maximum(out[pl.ds(g, 1), :], bm)
        return 0

    lax.fori_loop(0, NGRAPH, body, 0)


_pool = pl.pallas_call(
    _pool_body,
    grid=(NBLK,),
    in_specs=[pl.BlockSpec((RB, 1), lambda i: (i, 0))]
    + [pl.BlockSpec((RB, CQ), lambda i: (i, 0))] * 36,
    out_specs=pl.BlockSpec((NGRAPH, 9 * C), lambda i: (0, 0)),
    out_shape=jax.ShapeDtypeStruct((NGRAPH, 9 * C), jnp.float32),
)


def _head_body(p_ref, hW_ref, hb_ref, oW_ref, ob_ref, out_ref):
    z = jnp.maximum(
        jnp.dot(p_ref[...], hW_ref[...], preferred_element_type=jnp.float32)
        + hb_ref[...], 0.0)
    out_ref[...] = (jnp.dot(z, oW_ref[...], preferred_element_type=jnp.float32)
                    + ob_ref[...])


_head = pl.pallas_call(
    _head_body,
    out_shape=jax.ShapeDtypeStruct((NGRAPH, 1), jnp.float32),
)


def kernel(x, edge_index, batch, emb, conv_Wh, conv_bh, conv_Wo, conv_bo,
           hid_W, hid_b, out_W, out_b):
    src = edge_index[0].astype(jnp.int32)
    dst = edge_index[1].astype(jnp.int32)
    pad = jnp.full((EP - E,), PADI, jnp.int32)
    src2d = jnp.concatenate([src, pad]).reshape(NGRP, GIR, 128)
    dst2d = jnp.concatenate([dst, pad]).reshape(NGRP, GIR, 128)
    zeros = jnp.zeros((RPT, CQ), jnp.float32)

    xcol = x.reshape(N, 1).astype(jnp.int32)
    hq = _embed(xcol, emb)
    parts = list(hq)
    for i in range(8):
        segs = _segsum(*hq, src2d, dst2d, zeros)
        c2, sums = _mlp(*segs,
                        conv_Wh[i], conv_bh[i].reshape(1, HID),
                        conv_Wo[i], conv_bo[i].reshape(1, C))
        hq = _norm(*hq, c2, sums)
        parts.extend(hq)

    bcol = batch.reshape(N, 1).astype(jnp.int32)
    pooled = _pool(bcol, *parts)
    r = _head(pooled, hid_W, hid_b.reshape(1, -1), out_W, out_b.reshape(1, 1))
    return r.reshape(-1)


# speedup vs baseline: 6.0267x; 1.0957x over previous
import jax
import jax.numpy as jnp
from jax import lax
from jax.experimental import pallas as pl
from jax.experimental.pallas import tpu as pltpu
from jax.experimental.pallas import tpu_sc as plsc

N = 50000
E = 800000
C = 64
CQ = 16
NSUB = 16
RPT = 3128
NO = RPT * NSUB
PADI = NO
NPH = NO + 8
EPT = 50176
EP = EPT * NSUB
ROWS2D = EP // 128
GIR = 7
GB = GIR * 128
MH = 56
NGRP = ROWS2D // GIR
NRB = 4
NIB = 8
RB = 400
NBLK = N // RB
EPS = 1e-5
NGRAPH = 64
HID = 256
NEG = -jnp.inf


def _segsum_body(h0, h1, h2, h3, g0, g1, zeros,
                 o0, o1, o2, o3, b0, b1, b2, b3,
                 idx_g, idx_s, rows, acc, *allsems):
    cid = lax.axis_index("c")
    sid = lax.axis_index("s")
    hq = (h0, h1, h2, h3)
    oq = ((o0, o1, o2, o3), (b0, b1, b2, b3))
    semg = allsems[:NRB]
    sems = allsems[NRB:2 * NRB]
    semi = allsems[2 * NRB:]

    def do_pass(h_ref, o_ref, g3d, s3d):
        pltpu.sync_copy(zeros, acc.at[pl.ds(sid * RPT, RPT)])
        plsc.subcore_barrier()
        grp0 = sid * MH

        def load_idx(l, b):
            pltpu.async_copy(g3d.at[grp0 + b], idx_g.at[l], semi[l])
            pltpu.async_copy(s3d.at[grp0 + b], idx_s.at[l], semi[l])

        def wait_idx(l):
            pltpu.make_async_copy(g3d.at[0], idx_g.at[l], semi[l]).wait()
            pltpu.make_async_copy(s3d.at[0], idx_s.at[l], semi[l]).wait()

        def fire_g(r, l):
            for j in range(GIR):
                pltpu.async_copy(h_ref.at[idx_g.at[l, j]],
                                 rows.at[r, pl.ds(j * 128, 128)], semg[r])

        def fire_s(r, l):
            for j in range(GIR):
                pltpu.async_copy(rows.at[r, pl.ds(j * 128, 128)],
                                 acc.at[idx_s.at[l, j]], sems[r], add=True)

        def drain(sem):
            pltpu.make_async_copy(zeros.at[pl.ds(0, GB)],
                                  rows.at[0], sem).wait()

        for b in range(5):
            load_idx(b, b)
        wait_idx(0)
        fire_g(0, 0)
        wait_idx(1)
        fire_g(1, 1)

        def body(i, _):
            for s in range(8):
                m = 8 * i + s
                k4 = s % 4
                drain(semg[k4])
                fire_s(k4, s)

                @pl.when(m + 2 < MH)
                def _(m=m, s=s):
                    j4 = (s + 2) % 4
                    j8 = (s + 2) % 8

                    @pl.when(m >= 2)
                    def _():
                        drain(sems[j4])
                    wait_idx(j8)
                    fire_g(j4, j8)

                @pl.when(m + 5 < MH)
                def _(m=m, s=s):
                    load_idx((s + 5) % 8, m + 5)
            return 0

        lax.fori_loop(0, MH // 8, body, 0)
        for r in range(NRB):
            drain(sems[r])

        plsc.subcore_barrier()
        pltpu.sync_copy(acc.at[pl.ds(sid * RPT, RPT)],
                        o_ref.at[pl.ds(sid * RPT, RPT)])
        plsc.subcore_barrier()

    for p in range(2):
        g2d = g0 if p == 0 else g1
        s2d = g1 if p == 0 else g0
        for q in range(2):
            @pl.when(cid == 0)
            def _(p=p, q=q, g2d=g2d, s2d=s2d):
                do_pass(hq[q], oq[p][q], g2d, s2d)

            @pl.when(cid == 1)
            def _(p=p, q=q, g2d=g2d, s2d=s2d):
                do_pass(hq[2 + q], oq[p][2 + q], g2d, s2d)


_segsum = pl.kernel(
    _segsum_body,
    out_type=[jax.ShapeDtypeStruct((NO, CQ), jnp.float32)] * 8,
    mesh=plsc.VectorSubcoreMesh(core_axis_name="c", subcore_axis_name="s"),
    scratch_types=[
        pltpu.VMEM((NIB, GIR, 128), jnp.int32),
        pltpu.VMEM((NIB, GIR, 128), jnp.int32),
        pltpu.VMEM((NRB, GB, CQ), jnp.float32),
        pltpu.VMEM_SHARED((NPH, CQ), jnp.float32),
    ] + [pltpu.SemaphoreType.DMA] * (2 * NRB + NIB),
    compiler_params=pltpu.CompilerParams(use_tc_tiling_on_sc=False),
)


def _embed_body(xc_ref, emb_ref, *outs):
    xc = xc_ref[...]
    acc = jnp.zeros((RB, C), jnp.float32)
    for t in range(6):
        acc += jnp.where(xc == t, 1.0, 0.0) * emb_ref[t][None, :]
    for q in range(4):
        outs[q][...] = acc[:, q * CQ:(q + 1) * CQ]


_embed = pl.pallas_call(
    _embed_body,
    grid=(NBLK,),
    in_specs=[
        pl.BlockSpec((RB, 1), lambda i: (i, 0)),
        pl.BlockSpec((6, C), lambda i: (0, 0)),
    ],
    out_specs=[pl.BlockSpec((RB, CQ), lambda i: (i, 0))] * 4,
    out_shape=[jax.ShapeDtypeStruct((NPH, CQ), jnp.float32)] * 4,
)


def _mlp_body(*refs):
    outs = refs[:8]
    Wh, bh, Wo, bo = refs[8:12]
    c2_ref, s_ref = refs[12:14]
    i = pl.program_id(0)
    cc = jnp.concatenate([r[...] for r in outs], axis=1)
    hid = jnp.maximum(
        jnp.dot(cc, Wh[...], preferred_element_type=jnp.float32) + bh[...], 0.0)
    c2 = jnp.dot(hid, Wo[...], preferred_element_type=jnp.float32) + bo[...]
    c2_ref[...] = c2

    @pl.when(i == 0)
    def _():
        s_ref[...] = jnp.zeros_like(s_ref)

    s_ref[0:1, :] += jnp.sum(c2, axis=0, keepdims=True)
    s_ref[1:2, :] += jnp.sum(c2 * c2, axis=0, keepdims=True)


_mlp = pl.pallas_call(
    _mlp_body,
    grid=(NBLK,),
    in_specs=[pl.BlockSpec((RB, CQ), lambda i: (i, 0))] * 8 + [
        pl.BlockSpec((2 * C, HID), lambda i: (0, 0)),
        pl.BlockSpec((1, HID), lambda i: (0, 0)),
        pl.BlockSpec((HID, C), lambda i: (0, 0)),
        pl.BlockSpec((1, C), lambda i: (0, 0)),
    ],
    out_specs=[
        pl.BlockSpec((RB, C), lambda i: (i, 0)),
        pl.BlockSpec((8, C), lambda i: (0, 0)),
    ],
    out_shape=[
        jax.ShapeDtypeStruct((N, C), jnp.float32),
        jax.ShapeDtypeStruct((8, C), jnp.float32),
    ],
)


def _norm_body(*refs):
    hs = refs[:4]
    c2_ref, s_ref = refs[4:6]
    outs = refs[6:10]
    s = s_ref[...]
    mean = s[0:1, :] * (1.0 / N)
    var = s[1:2, :] * (1.0 / N) - mean * mean
    inv = lax.rsqrt(var + EPS)
    h = jnp.concatenate([r[...] for r in hs], axis=1)
    hn = jnp.maximum(h + (c2_ref[...] - mean) * inv, 0.0)
    for q in range(4):
        outs[q][...] = hn[:, q * CQ:(q + 1) * CQ]


_norm = pl.pallas_call(
    _norm_body,
    grid=(NBLK,),
    in_specs=[pl.BlockSpec((RB, CQ), lambda i: (i, 0))] * 4 + [
        pl.BlockSpec((RB, C), lambda i: (i, 0)),
        pl.BlockSpec((8, C), lambda i: (0, 0)),
    ],
    out_specs=[pl.BlockSpec((RB, CQ), lambda i: (i, 0))] * 4,
    out_shape=[jax.ShapeDtypeStruct((NPH, CQ), jnp.float32)] * 4,
)


def _pool_body(*refs):
    bcol_ref = refs[0]
    hs = refs[1:37]
    out = refs[37]
    i = pl.program_id(0)

    @pl.when(i == 0)
    def _():
        out[...] = jnp.full_like(out, NEG)

    xcat = jnp.concatenate([r[...] for r in hs], axis=1)
    b = bcol_ref[...]
    lo = jnp.min(b)
    hi = jnp.max(b)

    def body(g, _):
        @pl.when(jnp.logical_and(g >= lo, g <= hi))
        def _():
            bm = jnp.max(jnp.where(b == g, xcat, NEG), axis=0, keepdims=True)
            out[pl.ds(g, 1), :] = jnp.maximum(out[pl.ds(g, 1), :], bm)
        return 0

    lax.fori_loop(0, NGRAPH, body, 0)


_pool = pl.pallas_call(
    _pool_body,
    grid=(NBLK,),
    in_specs=[pl.BlockSpec((RB, 1), lambda i: (i, 0))]
    + [pl.BlockSpec((RB, CQ), lambda i: (i, 0))] * 36,
    out_specs=pl.BlockSpec((NGRAPH, 9 * C), lambda i: (0, 0)),
    out_shape=jax.ShapeDtypeStruct((NGRAPH, 9 * C), jnp.float32),
)


def _head_body(p_ref, hW_ref, hb_ref, oW_ref, ob_ref, out_ref):
    z = jnp.maximum(
        jnp.dot(p_ref[...], hW_ref[...], preferred_element_type=jnp.float32)
        + hb_ref[...], 0.0)
    out_ref[...] = (jnp.dot(z, oW_ref[...], preferred_element_type=jnp.float32)
                    + ob_ref[...])


_head = pl.pallas_call(
    _head_body,
    out_shape=jax.ShapeDtypeStruct((NGRAPH, 1), jnp.float32),
)


def kernel(x, edge_index, batch, emb, conv_Wh, conv_bh, conv_Wo, conv_bo,
           hid_W, hid_b, out_W, out_b):
    src = edge_index[0].astype(jnp.int32)
    dst = edge_index[1].astype(jnp.int32)
    pad = jnp.full((EP - E,), PADI, jnp.int32)
    src2d = jnp.concatenate([src, pad]).reshape(NGRP, GIR, 128)
    dst2d = jnp.concatenate([dst, pad]).reshape(NGRP, GIR, 128)
    zeros = jnp.zeros((RPT, CQ), jnp.float32)

    xcol = x.reshape(N, 1).astype(jnp.int32)
    hq = _embed(xcol, emb)
    parts = list(hq)
    for i in range(8):
        segs = _segsum(*hq, src2d, dst2d, zeros)
        c2, sums = _mlp(*segs,
                        conv_Wh[i], conv_bh[i].reshape(1, HID),
                        conv_Wo[i], conv_bo[i].reshape(1, C))
        hq = _norm(*hq, c2, sums)
        parts.extend(hq)

    bcol = batch.reshape(N, 1).astype(jnp.int32)
    pooled = _pool(bcol, *parts)
    r = _head(pooled, hid_W, hid_b.reshape(1, -1), out_W, out_b.reshape(1, 1))
    return r.reshape(-1)
